# baseline (speedup 1.0000x reference)
import functools

import jax
import jax.numpy as jnp
from jax import lax
from jax.experimental import pallas as pl
from jax.experimental.pallas import tpu as pltpu
from jax.experimental.pallas import tpu_sc as plsc

T = 2048
H = 16
D = 128
ROPE_DIM = 64
TOPK = 512
EPS = 1e-6

TB = 256
SB = 512

NC = 2
NS = 16
NW = NC * NS
RPW = T // NW


def _scores_body(qq_ref, k_ref, w_ref, out_ref):
    t = pl.program_id(0)
    s = pl.program_id(1)
    fully_masked = (t + 1) * TB - 1 < s * SB

    @pl.when(fully_masked)
    def _():
        out_ref[...] = jnp.full((TB, SB), -1e30, dtype=jnp.float32)

    @pl.when(jnp.logical_not(fully_masked))
    def _():
        w_blk = w_ref[...]
        k_blk = k_ref[...]
        acc = jnp.zeros((TB, SB), jnp.float32)
        for h in range(H):
            logits = jax.lax.dot_general(
                qq_ref[:, h * D:(h + 1) * D], k_blk,
                (((1,), (1,)), ((), ())),
                preferred_element_type=jnp.float32)
            acc = acc + w_blk[:, h:h + 1] * jnp.maximum(logits, 0.0)
        rows = t * TB + jax.lax.broadcasted_iota(jnp.int32, (TB, SB), 0)
        cols = s * SB + jax.lax.broadcasted_iota(jnp.int32, (TB, SB), 1)
        out_ref[...] = jnp.where(rows < cols, -1e30, acc)


def _scores_call(qq, k, w):
    return pl.pallas_call(
        _scores_body,
        grid=(T // TB, T // SB),
        in_specs=[
            pl.BlockSpec((TB, H * D), lambda t, s: (t, 0)),
            pl.BlockSpec((SB, D), lambda t, s: (s, 0)),
            pl.BlockSpec((TB, H), lambda t, s: (t, 0)),
        ],
        out_specs=pl.BlockSpec((TB, SB), lambda t, s: (t, s)),
        out_shape=jax.ShapeDtypeStruct((T, T), jnp.float32),
    )(qq, k, w)


def _bisect_body(s_ref, thr_ref, cnt_ref):
    bits = jax.lax.bitcast_convert_type(s_ref[...], jnp.int32)
    key = jnp.where(bits >= 0, bits, bits ^ jnp.int32(0x7FFFFFFF))

    def it(_, lohi):
        lo, hi = lohi
        mid = (lo | hi) - ((lo ^ hi) >> 1)
        cnt = jnp.sum((key >= mid).astype(jnp.int32), axis=1, keepdims=True)
        ok = cnt >= TOPK
        return jnp.where(ok, mid, lo), jnp.where(ok, hi, mid - 1)

    init = (jnp.full((TB, 1), -2**31, jnp.int32),
            jnp.full((TB, 1), 2**31 - 1, jnp.int32))
    lo, _ = jax.lax.fori_loop(0, 32, it, init)
    cntgt = jnp.sum((key > lo).astype(jnp.int32), axis=1, keepdims=True)
    thr_ref[...] = jnp.broadcast_to(lo, (TB, 128))
    cnt_ref[...] = jnp.broadcast_to(cntgt, (TB, 128))


def _bisect_call(scores):
    return pl.pallas_call(
        _bisect_body,
        grid=(T // TB,),
        in_specs=[pl.BlockSpec((TB, T), lambda t: (t, 0))],
        out_specs=[pl.BlockSpec((TB, 128), lambda t: (t, 0)),
                   pl.BlockSpec((TB, 128), lambda t: (t, 0))],
        out_shape=[jax.ShapeDtypeStruct((T, 128), jnp.int32),
                   jax.ShapeDtypeStruct((T, 128), jnp.int32)],
    )(scores)


def _sc_topk_body(scores_hbm, thr_hbm, cnt_hbm, vals_hbm, idx_hbm,
                  row_v, thr_v, cnt_v, kc, ic, kd, idd, bins, vout):
    wid = lax.axis_index("s") * NC + lax.axis_index("c")
    base = wid * RPW
    pltpu.sync_copy(thr_hbm.at[pl.ds(base, RPW)], thr_v)
    pltpu.sync_copy(cnt_hbm.at[pl.ds(base, RPW)], cnt_v)
    iota = jax.lax.broadcasted_iota(jnp.int32, (16,), 0)
    ones = jnp.ones((16,), jnp.int32)
    zeros = jnp.zeros((16,), jnp.int32)
    magic = jnp.full((16,), 0x7FFFFFFF, jnp.int32)

    def radix_pass(shift, src_k, src_i, dst_k, dst_i):
        def zero_body(i, c):
            plsc.store_scatter(bins, [iota + i * 16], zeros)
            return c
        lax.fori_loop(0, 32, zero_body, 0)

        def hist_body(v, c):
            pos = iota * 32 + v
            kk = plsc.load_gather(src_k, [pos])
            dig = jax.lax.shift_right_logical(kk, shift) & 31
            plsc.addupdate_scatter(bins, [dig * 16 + iota], ones)
            return c
        lax.fori_loop(0, 32, hist_body, 0)

        def scan_body(i, run):
            bidx = iota + i * 16
            bcur = plsc.load_gather(bins, [bidx])
            incl = plsc.cumsum(bcur)
            plsc.store_scatter(bins, [bidx], incl - bcur + run)
            return run + jnp.sum(bcur)
        lax.fori_loop(0, 32, scan_body, jnp.int32(0))

        def perm_body(v, c):
            pos = iota * 32 + v
            kk = plsc.load_gather(src_k, [pos])
            ii = plsc.load_gather(src_i, [pos])
            addr = (jax.lax.shift_right_logical(kk, shift) & 31) * 16 + iota
            off = plsc.load_gather(bins, [addr])
            plsc.store_scatter(dst_k, [off], kk)
            plsc.store_scatter(dst_i, [off], ii)
            plsc.store_scatter(bins, [addr], off + 1)
            return c
        lax.fori_loop(0, 32, perm_body, 0)

    def row_body(r, carry):
        row = base + r
        pltpu.sync_copy(scores_hbm.at[row], row_v)
        rsplat = zeros + r
        th = plsc.load_gather(thr_v, [rsplat])
        eq_need = 512 - plsc.load_gather(cnt_v, [rsplat])

        def compact_body(v, st):
            off, eqs = st
            pos = iota + v * 16
            b = plsc.bitcast(plsc.load_gather(row_v, [pos]), jnp.int32)
            key = jnp.where(b >= 0, b, b ^ magic)
            m_gt = key > th
            m_eq = key == th
            meqi = m_eq.astype(jnp.int32)
            eq_before = eqs + (plsc.cumsum(meqi) - meqi)
            keep = jnp.logical_or(
                m_gt, jnp.logical_and(m_eq, eq_before < eq_need))
            ki = keep.astype(jnp.int32)
            dest = off + (plsc.cumsum(ki) - ki)
            plsc.store_scatter(kc, [dest], magic - key, mask=keep)
            plsc.store_scatter(ic, [dest], pos, mask=keep)
            return (off + plsc.all_reduce_population_count(keep),
                    eqs + plsc.all_reduce_population_count(m_eq))
        lax.fori_loop(0, 128, compact_body, (zeros, zeros))

        radix_pass(0, kc, ic, kd, idd)
        radix_pass(5, kd, idd, kc, ic)
        radix_pass(10, kc, ic, kd, idd)
        radix_pass(15, kd, idd, kc, ic)
        radix_pass(20, kc, ic, kd, idd)
        radix_pass(25, kd, idd, kc, ic)
        radix_pass(30, kc, ic, kd, idd)

        def out_body(i, c):
            posi = iota + i * 16
            key = magic - plsc.load_gather(kd, [posi])
            bb = jnp.where(key >= 0, key, key ^ magic)
            plsc.store_scatter(vout, [posi], plsc.bitcast(bb, jnp.float32))
            return c
        lax.fori_loop(0, 32, out_body, 0)

        pltpu.sync_copy(vout, vals_hbm.at[row])
        pltpu.sync_copy(idd, idx_hbm.at[row])
        return carry

    lax.fori_loop(0, RPW, row_body, 0)


_sc_topk = functools.partial(
    pl.kernel,
    _sc_topk_body,
    out_type=[jax.ShapeDtypeStruct((T, TOPK), jnp.float32),
              jax.ShapeDtypeStruct((T, TOPK), jnp.int32)],
    mesh=plsc.VectorSubcoreMesh(core_axis_name="c", subcore_axis_name="s"),
    compiler_params=pltpu.CompilerParams(needs_layout_passes=False),
    scratch_types=[
        pltpu.VMEM((T,), jnp.float32),
        pltpu.VMEM((RPW,), jnp.int32),
        pltpu.VMEM((RPW,), jnp.int32),
        pltpu.VMEM((TOPK,), jnp.int32),
        pltpu.VMEM((TOPK,), jnp.int32),
        pltpu.VMEM((TOPK,), jnp.int32),
        pltpu.VMEM((TOPK,), jnp.int32),
        pltpu.VMEM((TOPK,), jnp.int32),
        pltpu.VMEM((TOPK,), jnp.float32),
    ],
)()


def _rope(x, cos, sin):
    half = x.shape[-1] // 2
    x1 = x[..., :half]
    x2 = x[..., half:]
    return jnp.concatenate([x1 * cos - x2 * sin, x2 * cos + x1 * sin], axis=-1)


def _compute_scores(hidden_states, q_lora, wq_b, wk, k_norm_w, k_norm_b,
                    w_proj, cos_cache, sin_cache, positions):
    q = (q_lora @ wq_b).reshape(T, H, D)
    k = hidden_states @ wk
    mu = jnp.mean(k, axis=-1, keepdims=True)
    var = jnp.var(k, axis=-1, keepdims=True)
    k = (k - mu) / jnp.sqrt(var + EPS) * k_norm_w + k_norm_b
    cos = jnp.take(cos_cache, positions, axis=0)
    sin = jnp.take(sin_cache, positions, axis=0)
    q_rot = _rope(q[..., :ROPE_DIM], cos[:, None, :], sin[:, None, :])
    q = jnp.concatenate([q_rot, q[..., ROPE_DIM:]], axis=-1)
    k_rot = _rope(k[..., :ROPE_DIM], cos, sin)
    k = jnp.concatenate([k_rot, k[..., ROPE_DIM:]], axis=-1)
    q_scale = jnp.max(jnp.abs(q), axis=-1, keepdims=True) / 448.0 + 1e-12
    q_q = q / q_scale
    softmax_scale = D ** (-0.5)
    weights_scale = H ** (-0.5)
    w = hidden_states @ w_proj
    w = w * q_scale[:, :, 0] * (softmax_scale * weights_scale)
    return _scores_call(q_q.reshape(T, H * D), k, w)


def kernel(hidden_states, q_lora, wq_b, wk, k_norm_w, k_norm_b, w_proj,
           cos_cache, sin_cache, positions):
    scores = _compute_scores(hidden_states, q_lora, wq_b, wk, k_norm_w,
                             k_norm_b, w_proj, cos_cache, sin_cache,
                             positions)
    thr2d, cnt2d = _bisect_call(scores)
    vals, idx = _sc_topk(scores, thr2d[:, 0], cnt2d[:, 0])
    return vals, idx


# speedup vs baseline: 1.6361x; 1.6361x over previous
import functools

import jax
import jax.numpy as jnp
from jax import lax
from jax.experimental import pallas as pl
from jax.experimental.pallas import tpu as pltpu
from jax.experimental.pallas import tpu_sc as plsc

T = 2048
H = 16
D = 128
ROPE_DIM = 64
TOPK = 512
EPS = 1e-6

TB = 256
SB = 512

NC = 2
NS = 16
NW = NC * NS
RPW = T // NW


def _scores_body(qq_ref, k_ref, w_ref, out_ref):
    t = pl.program_id(0)
    s = pl.program_id(1)
    fully_masked = (t + 1) * TB - 1 < s * SB

    @pl.when(fully_masked)
    def _():
        out_ref[...] = jnp.full((TB, SB), -1e30, dtype=jnp.float32)

    @pl.when(jnp.logical_not(fully_masked))
    def _():
        w_blk = w_ref[...]
        k_blk = k_ref[...]
        acc = jnp.zeros((TB, SB), jnp.float32)
        for h in range(H):
            logits = jax.lax.dot_general(
                qq_ref[:, h * D:(h + 1) * D], k_blk,
                (((1,), (1,)), ((), ())),
                preferred_element_type=jnp.float32)
            acc = acc + w_blk[:, h:h + 1] * jnp.maximum(logits, 0.0)
        rows = t * TB + jax.lax.broadcasted_iota(jnp.int32, (TB, SB), 0)
        cols = s * SB + jax.lax.broadcasted_iota(jnp.int32, (TB, SB), 1)
        out_ref[...] = jnp.where(rows < cols, -1e30, acc)


def _scores_call(qq, k, w):
    return pl.pallas_call(
        _scores_body,
        grid=(T // TB, T // SB),
        in_specs=[
            pl.BlockSpec((TB, H * D), lambda t, s: (t, 0)),
            pl.BlockSpec((SB, D), lambda t, s: (s, 0)),
            pl.BlockSpec((TB, H), lambda t, s: (t, 0)),
        ],
        out_specs=pl.BlockSpec((TB, SB), lambda t, s: (t, s)),
        out_shape=jax.ShapeDtypeStruct((T, T), jnp.float32),
    )(qq, k, w)


def _bisect_body(s_ref, thr_ref, cnt_ref):
    bits = jax.lax.bitcast_convert_type(s_ref[...], jnp.int32)
    key = jnp.where(bits >= 0, bits, bits ^ jnp.int32(0x7FFFFFFF))

    def it(_, lohi):
        lo, hi = lohi
        mid = (lo | hi) - ((lo ^ hi) >> 1)
        cnt = jnp.sum((key >= mid).astype(jnp.int32), axis=1, keepdims=True)
        ok = cnt >= TOPK
        return jnp.where(ok, mid, lo), jnp.where(ok, hi, mid - 1)

    init = (jnp.full((TB, 1), -2**31, jnp.int32),
            jnp.full((TB, 1), 2**31 - 1, jnp.int32))
    lo, _ = jax.lax.fori_loop(0, 32, it, init)
    cntgt = jnp.sum((key > lo).astype(jnp.int32), axis=1, keepdims=True)
    thr_ref[...] = jnp.broadcast_to(lo, (TB, 128))
    cnt_ref[...] = jnp.broadcast_to(cntgt, (TB, 128))


def _bisect_call(scores):
    return pl.pallas_call(
        _bisect_body,
        grid=(T // TB,),
        in_specs=[pl.BlockSpec((TB, T), lambda t: (t, 0))],
        out_specs=[pl.BlockSpec((TB, 128), lambda t: (t, 0)),
                   pl.BlockSpec((TB, 128), lambda t: (t, 0))],
        out_shape=[jax.ShapeDtypeStruct((T, 128), jnp.int32),
                   jax.ShapeDtypeStruct((T, 128), jnp.int32)],
    )(scores)


def _sc_topk_body(scores_hbm, thr_hbm, cnt_hbm, vals_hbm, idx_hbm,
                  row_v, thr_v, cnt_v, ic, vout):
    wid = lax.axis_index("s") * NC + lax.axis_index("c")
    base = wid * RPW
    pltpu.sync_copy(thr_hbm.at[pl.ds(base, RPW)], thr_v)
    pltpu.sync_copy(cnt_hbm.at[pl.ds(base, RPW)], cnt_v)
    iota = jax.lax.broadcasted_iota(jnp.int32, (16,), 0)
    ones = jnp.ones((16,), jnp.int32)
    zeros = jnp.zeros((16,), jnp.int32)
    magic = jnp.full((16,), 0x7FFFFFFF, jnp.int32)

    def row_body(r, carry):
        row = base + r
        pltpu.sync_copy(scores_hbm.at[row], row_v)
        rsplat = zeros + r
        th = plsc.load_gather(thr_v, [rsplat])
        eq_need = 512 - plsc.load_gather(cnt_v, [rsplat])

        def compact_body(v, st):
            off, eqs = st
            pos = iota + v * 16
            x = plsc.load_gather(row_v, [pos])
            b = plsc.bitcast(x, jnp.int32)
            key = jnp.where(b >= 0, b, b ^ magic)
            m_gt = key > th
            m_eq = key == th
            meqi = m_eq.astype(jnp.int32)
            eq_before = eqs + (plsc.cumsum(meqi) - meqi)
            keep = jnp.logical_or(
                m_gt, jnp.logical_and(m_eq, eq_before < eq_need))
            ki = keep.astype(jnp.int32)
            dest = off + (plsc.cumsum(ki) - ki)
            plsc.store_scatter(vout, [dest], x, mask=keep)
            plsc.store_scatter(ic, [dest], pos, mask=keep)
            return (off + plsc.all_reduce_population_count(keep),
                    eqs + plsc.all_reduce_population_count(m_eq))
        lax.fori_loop(0, 128, compact_body, (zeros, zeros))

        pltpu.sync_copy(vout, vals_hbm.at[row])
        pltpu.sync_copy(ic, idx_hbm.at[row])
        return carry

    lax.fori_loop(0, RPW, row_body, 0)


_sc_topk = functools.partial(
    pl.kernel,
    _sc_topk_body,
    out_type=[jax.ShapeDtypeStruct((T, TOPK), jnp.float32),
              jax.ShapeDtypeStruct((T, TOPK), jnp.int32)],
    mesh=plsc.VectorSubcoreMesh(core_axis_name="c", subcore_axis_name="s"),
    compiler_params=pltpu.CompilerParams(needs_layout_passes=False),
    scratch_types=[
        pltpu.VMEM((T,), jnp.float32),
        pltpu.VMEM((RPW,), jnp.int32),
        pltpu.VMEM((RPW,), jnp.int32),
        pltpu.VMEM((TOPK,), jnp.int32),
        pltpu.VMEM((TOPK,), jnp.float32),
    ],
)()


def _sort_body(v_ref, i_ref, vo_ref, io_ref):
    v = v_ref[...]
    ix = i_ref[...]
    lane = jax.lax.broadcasted_iota(jnp.int32, (TB, TOPK), 1)
    k = 2
    while k <= TOPK:
        dirdesc = (lane & k) == 0 if k < TOPK else lane >= 0
        j = k // 2
        while j >= 1:
            low = (lane & j) == 0
            pv = jnp.where(low, jnp.roll(v, -j, axis=1), jnp.roll(v, j, axis=1))
            pix = jnp.where(low, jnp.roll(ix, -j, axis=1),
                            jnp.roll(ix, j, axis=1))
            beats = jnp.logical_or(
                v > pv, jnp.logical_and(v == pv, ix < pix))
            keep = beats == (low == dirdesc)
            v = jnp.where(keep, v, pv)
            ix = jnp.where(keep, ix, pix)
            j //= 2
        k *= 2
    vo_ref[...] = v
    io_ref[...] = ix


def _sort_call(vals_c, idx_c):
    return pl.pallas_call(
        _sort_body,
        grid=(T // TB,),
        in_specs=[pl.BlockSpec((TB, TOPK), lambda t: (t, 0)),
                  pl.BlockSpec((TB, TOPK), lambda t: (t, 0))],
        out_specs=[pl.BlockSpec((TB, TOPK), lambda t: (t, 0)),
                   pl.BlockSpec((TB, TOPK), lambda t: (t, 0))],
        out_shape=[jax.ShapeDtypeStruct((T, TOPK), jnp.float32),
                   jax.ShapeDtypeStruct((T, TOPK), jnp.int32)],
    )(vals_c, idx_c)


def _rope(x, cos, sin):
    half = x.shape[-1] // 2
    x1 = x[..., :half]
    x2 = x[..., half:]
    return jnp.concatenate([x1 * cos - x2 * sin, x2 * cos + x1 * sin], axis=-1)


def _compute_scores(hidden_states, q_lora, wq_b, wk, k_norm_w, k_norm_b,
                    w_proj, cos_cache, sin_cache, positions):
    q = (q_lora @ wq_b).reshape(T, H, D)
    k = hidden_states @ wk
    mu = jnp.mean(k, axis=-1, keepdims=True)
    var = jnp.var(k, axis=-1, keepdims=True)
    k = (k - mu) / jnp.sqrt(var + EPS) * k_norm_w + k_norm_b
    cos = jnp.take(cos_cache, positions, axis=0)
    sin = jnp.take(sin_cache, positions, axis=0)
    q_rot = _rope(q[..., :ROPE_DIM], cos[:, None, :], sin[:, None, :])
    q = jnp.concatenate([q_rot, q[..., ROPE_DIM:]], axis=-1)
    k_rot = _rope(k[..., :ROPE_DIM], cos, sin)
    k = jnp.concatenate([k_rot, k[..., ROPE_DIM:]], axis=-1)
    q_scale = jnp.max(jnp.abs(q), axis=-1, keepdims=True) / 448.0 + 1e-12
    q_q = q / q_scale
    softmax_scale = D ** (-0.5)
    weights_scale = H ** (-0.5)
    w = hidden_states @ w_proj
    w = w * q_scale[:, :, 0] * (softmax_scale * weights_scale)
    return _scores_call(q_q.reshape(T, H * D), k, w)


def kernel(hidden_states, q_lora, wq_b, wk, k_norm_w, k_norm_b, w_proj,
           cos_cache, sin_cache, positions):
    scores = _compute_scores(hidden_states, q_lora, wq_b, wk, k_norm_w,
                             k_norm_b, w_proj, cos_cache, sin_cache,
                             positions)
    thr2d, cnt2d = _bisect_call(scores)
    vals_c, idx_c = _sc_topk(scores, thr2d[:, 0], cnt2d[:, 0])
    return _sort_call(vals_c, idx_c)


# speedup vs baseline: 1.6412x; 1.0031x over previous
import functools

import jax
import jax.numpy as jnp
from jax import lax
from jax.experimental import pallas as pl
from jax.experimental.pallas import tpu as pltpu
from jax.experimental.pallas import tpu_sc as plsc

T = 2048
H = 16
D = 128
ROPE_DIM = 64
TOPK = 512
EPS = 1e-6

TB = 256
SB = 512

NC = 2
NS = 16
NW = NC * NS
RPW = T // NW


def _scores_body(qq_ref, k_ref, w_ref, out_ref):
    t = pl.program_id(0)
    s = pl.program_id(1)
    fully_masked = (t + 1) * TB - 1 < s * SB

    @pl.when(fully_masked)
    def _():
        out_ref[...] = jnp.full((TB, SB), -1e30, dtype=jnp.float32)

    @pl.when(jnp.logical_not(fully_masked))
    def _():
        w_blk = w_ref[...]
        k_blk = k_ref[...]
        acc = jnp.zeros((TB, SB), jnp.float32)
        for h in range(H):
            logits = jax.lax.dot_general(
                qq_ref[:, h * D:(h + 1) * D], k_blk,
                (((1,), (1,)), ((), ())),
                preferred_element_type=jnp.float32)
            acc = acc + w_blk[:, h:h + 1] * jnp.maximum(logits, 0.0)
        rows = t * TB + jax.lax.broadcasted_iota(jnp.int32, (TB, SB), 0)
        cols = s * SB + jax.lax.broadcasted_iota(jnp.int32, (TB, SB), 1)
        out_ref[...] = jnp.where(rows < cols, -1e30, acc)


def _scores_call(qq, k, w):
    return pl.pallas_call(
        _scores_body,
        grid=(T // TB, T // SB),
        in_specs=[
            pl.BlockSpec((TB, H * D), lambda t, s: (t, 0)),
            pl.BlockSpec((SB, D), lambda t, s: (s, 0)),
            pl.BlockSpec((TB, H), lambda t, s: (t, 0)),
        ],
        out_specs=pl.BlockSpec((TB, SB), lambda t, s: (t, s)),
        out_shape=jax.ShapeDtypeStruct((T, T), jnp.float32),
    )(qq, k, w)


def _bisect_body(s_ref, thr_ref, cnt_ref):
    bits = jax.lax.bitcast_convert_type(s_ref[...], jnp.int32)
    key = jnp.where(bits >= 0, bits, bits ^ jnp.int32(0x7FFFFFFF))

    def it(_, lohi):
        lo, hi = lohi
        mid = (lo | hi) - ((lo ^ hi) >> 1)
        cnt = jnp.sum((key >= mid).astype(jnp.int32), axis=1, keepdims=True)
        ok = cnt >= TOPK
        return jnp.where(ok, mid, lo), jnp.where(ok, hi, mid - 1)

    init = (jnp.full((TB, 1), -2**31, jnp.int32),
            jnp.full((TB, 1), 2**31 - 1, jnp.int32))
    lo, _ = jax.lax.fori_loop(0, 32, it, init)
    cntgt = jnp.sum((key > lo).astype(jnp.int32), axis=1, keepdims=True)
    thr_ref[...] = jnp.broadcast_to(lo, (TB, 128))
    cnt_ref[...] = jnp.broadcast_to(cntgt, (TB, 128))


def _bisect_call(scores):
    return pl.pallas_call(
        _bisect_body,
        grid=(T // TB,),
        in_specs=[pl.BlockSpec((TB, T), lambda t: (t, 0))],
        out_specs=[pl.BlockSpec((TB, 128), lambda t: (t, 0)),
                   pl.BlockSpec((TB, 128), lambda t: (t, 0))],
        out_shape=[jax.ShapeDtypeStruct((T, 128), jnp.int32),
                   jax.ShapeDtypeStruct((T, 128), jnp.int32)],
    )(scores)


def _sc_topk_body(scores_hbm, thr_hbm, cnt_hbm, vals_hbm, idx_hbm,
                  row_v, thr_v, cnt_v, ic, vout):
    wid = lax.axis_index("s") * NC + lax.axis_index("c")
    base = wid * RPW
    pltpu.sync_copy(thr_hbm.at[pl.ds(base, RPW)], thr_v)
    pltpu.sync_copy(cnt_hbm.at[pl.ds(base, RPW)], cnt_v)
    iota = jax.lax.broadcasted_iota(jnp.int32, (16,), 0)
    ones = jnp.ones((16,), jnp.int32)
    zeros = jnp.zeros((16,), jnp.int32)
    magic = jnp.full((16,), 0x7FFFFFFF, jnp.int32)

    def row_body(r, carry):
        row = base + r
        pltpu.sync_copy(scores_hbm.at[row], row_v)
        rsplat = zeros + r
        th = plsc.load_gather(thr_v, [rsplat])
        eq_need = 512 - plsc.load_gather(cnt_v, [rsplat])

        def compact_body(v, st):
            off, eqs = st
            for sub in range(4):
                pos = iota + (v * 4 + sub) * 16
                x = plsc.load_gather(row_v, [pos])
                b = plsc.bitcast(x, jnp.int32)
                key = jnp.where(b >= 0, b, b ^ magic)
                m_gt = key > th
                m_eq = key == th
                meqi = m_eq.astype(jnp.int32)
                eq_before = eqs + (plsc.cumsum(meqi) - meqi)
                keep = jnp.logical_or(
                    m_gt, jnp.logical_and(m_eq, eq_before < eq_need))
                ki = keep.astype(jnp.int32)
                dest = off + (plsc.cumsum(ki) - ki)
                plsc.store_scatter(vout, [dest], x, mask=keep)
                plsc.store_scatter(ic, [dest], pos, mask=keep)
                off = off + plsc.all_reduce_population_count(keep)
                eqs = eqs + plsc.all_reduce_population_count(m_eq)
            return (off, eqs)
        lax.fori_loop(0, 32, compact_body, (zeros, zeros))

        pltpu.sync_copy(vout, vals_hbm.at[row])
        pltpu.sync_copy(ic, idx_hbm.at[row])
        return carry

    lax.fori_loop(0, RPW, row_body, 0)


_sc_topk = functools.partial(
    pl.kernel,
    _sc_topk_body,
    out_type=[jax.ShapeDtypeStruct((T, TOPK), jnp.float32),
              jax.ShapeDtypeStruct((T, TOPK), jnp.int32)],
    mesh=plsc.VectorSubcoreMesh(core_axis_name="c", subcore_axis_name="s"),
    compiler_params=pltpu.CompilerParams(needs_layout_passes=False),
    scratch_types=[
        pltpu.VMEM((T,), jnp.float32),
        pltpu.VMEM((RPW,), jnp.int32),
        pltpu.VMEM((RPW,), jnp.int32),
        pltpu.VMEM((TOPK,), jnp.int32),
        pltpu.VMEM((TOPK,), jnp.float32),
    ],
)()


def _sort_body(v_ref, i_ref, vo_ref, io_ref):
    v = v_ref[...]
    ix = i_ref[...]
    lane = jax.lax.broadcasted_iota(jnp.int32, (TB, TOPK), 1)
    k = 2
    while k <= TOPK:
        dirdesc = (lane & k) == 0 if k < TOPK else lane >= 0
        j = k // 2
        while j >= 1:
            low = (lane & j) == 0
            pv = jnp.where(low, jnp.roll(v, -j, axis=1), jnp.roll(v, j, axis=1))
            pix = jnp.where(low, jnp.roll(ix, -j, axis=1),
                            jnp.roll(ix, j, axis=1))
            beats = jnp.logical_or(
                v > pv, jnp.logical_and(v == pv, ix < pix))
            keep = beats == (low == dirdesc)
            v = jnp.where(keep, v, pv)
            ix = jnp.where(keep, ix, pix)
            j //= 2
        k *= 2
    vo_ref[...] = v
    io_ref[...] = ix


def _sort_call(vals_c, idx_c):
    return pl.pallas_call(
        _sort_body,
        grid=(T // TB,),
        in_specs=[pl.BlockSpec((TB, TOPK), lambda t: (t, 0)),
                  pl.BlockSpec((TB, TOPK), lambda t: (t, 0))],
        out_specs=[pl.BlockSpec((TB, TOPK), lambda t: (t, 0)),
                   pl.BlockSpec((TB, TOPK), lambda t: (t, 0))],
        out_shape=[jax.ShapeDtypeStruct((T, TOPK), jnp.float32),
                   jax.ShapeDtypeStruct((T, TOPK), jnp.int32)],
    )(vals_c, idx_c)


def _rope(x, cos, sin):
    half = x.shape[-1] // 2
    x1 = x[..., :half]
    x2 = x[..., half:]
    return jnp.concatenate([x1 * cos - x2 * sin, x2 * cos + x1 * sin], axis=-1)


def _compute_scores(hidden_states, q_lora, wq_b, wk, k_norm_w, k_norm_b,
                    w_proj, cos_cache, sin_cache, positions):
    q = (q_lora @ wq_b).reshape(T, H, D)
    k = hidden_states @ wk
    mu = jnp.mean(k, axis=-1, keepdims=True)
    var = jnp.var(k, axis=-1, keepdims=True)
    k = (k - mu) / jnp.sqrt(var + EPS) * k_norm_w + k_norm_b
    cos = jnp.take(cos_cache, positions, axis=0)
    sin = jnp.take(sin_cache, positions, axis=0)
    q_rot = _rope(q[..., :ROPE_DIM], cos[:, None, :], sin[:, None, :])
    q = jnp.concatenate([q_rot, q[..., ROPE_DIM:]], axis=-1)
    k_rot = _rope(k[..., :ROPE_DIM], cos, sin)
    k = jnp.concatenate([k_rot, k[..., ROPE_DIM:]], axis=-1)
    q_scale = jnp.max(jnp.abs(q), axis=-1, keepdims=True) / 448.0 + 1e-12
    q_q = q / q_scale
    softmax_scale = D ** (-0.5)
    weights_scale = H ** (-0.5)
    w = hidden_states @ w_proj
    w = w * q_scale[:, :, 0] * (softmax_scale * weights_scale)
    return _scores_call(q_q.reshape(T, H * D), k, w)


def kernel(hidden_states, q_lora, wq_b, wk, k_norm_w, k_norm_b, w_proj,
           cos_cache, sin_cache, positions):
    scores = _compute_scores(hidden_states, q_lora, wq_b, wk, k_norm_w,
                             k_norm_b, w_proj, cos_cache, sin_cache,
                             positions)
    thr2d, cnt2d = _bisect_call(scores)
    vals_c, idx_c = _sc_topk(scores, thr2d[:, 0], cnt2d[:, 0])
    return _sort_call(vals_c, idx_c)
